# Initial kernel scaffold; baseline (speedup 1.0000x reference)
import jax
import jax.numpy as jnp
from jax.experimental import pallas as pl


def kernel(x, edge_index, edge_attr, batch, W1, b1, W2, b2, Wf1, bf1, Wf2, bf2):
    raise NotImplementedError("write your pallas kernel here")


# speedup vs baseline: 18.0000x; 18.0000x over previous
import functools

import jax
import jax.numpy as jnp
from jax import lax
from jax.experimental import pallas as pl
from jax.experimental.pallas import tpu as pltpu
from jax.experimental.pallas import tpu_sc as plsc

N = 10000
E = 320000
DIN = 128
H = 64
G = 64

NC = 2
NS = 16
CHUNK = 128
NCHUNKS = E // CHUNK
CPC = NCHUNKS // NC
BASE_TRIPS = CPC // NS
EXTRA = CPC - BASE_TRIPS * NS
SLICE = 640
LAST_SLICE = N - (NS - 1) * SLICE

_mesh = plsc.VectorSubcoreMesh(core_axis_name="c", subcore_axis_name="s")


def _sc_deg(col, ones128, zcol):
    def body(col_hbm, ones_hbm, z_hbm, deg0_hbm, deg1_hbm, idx_v, ones_v,
             slab_v, acc_sh):
        cid = lax.axis_index("c")
        sid = lax.axis_index("s")
        pltpu.sync_copy(ones_hbm, ones_v)
        pltpu.sync_copy(z_hbm, slab_v)

        @pl.when(sid < NS - 1)
        def _():
            pltpu.sync_copy(slab_v, acc_sh.at[pl.ds(sid * SLICE, SLICE)])

        @pl.when(sid == NS - 1)
        def _():
            pltpu.sync_copy(slab_v.at[pl.ds(0, LAST_SLICE)],
                            acc_sh.at[pl.ds(sid * SLICE, LAST_SLICE)])

        plsc.subcore_barrier()
        ntrips = jnp.where(sid < EXTRA, BASE_TRIPS + 1, BASE_TRIPS)

        def trip(k, c):
            g = cid * CPC + sid + k * NS
            base = g * CHUNK
            pltpu.sync_copy(col_hbm.at[pl.ds(base, CHUNK)], idx_v)
            pltpu.sync_copy(ones_v, acc_sh.at[idx_v], add=True)
            return c

        lax.fori_loop(0, ntrips, trip, 0)
        plsc.subcore_barrier()
        for c, dst in ((0, deg0_hbm), (1, deg1_hbm)):
            @pl.when(cid == c)
            def _(dst=dst):
                @pl.when(sid < NS - 1)
                def _():
                    pltpu.sync_copy(acc_sh.at[pl.ds(sid * SLICE, SLICE)],
                                    slab_v)
                    pltpu.sync_copy(slab_v, dst.at[pl.ds(sid * SLICE, SLICE)])

                @pl.when(sid == NS - 1)
                def _():
                    pltpu.sync_copy(acc_sh.at[pl.ds(sid * SLICE, LAST_SLICE)],
                                    slab_v.at[pl.ds(0, LAST_SLICE)])
                    pltpu.sync_copy(slab_v.at[pl.ds(0, LAST_SLICE)],
                                    dst.at[pl.ds(sid * SLICE, LAST_SLICE)])

    f = pl.kernel(
        body,
        out_type=[jax.ShapeDtypeStruct((N,), jnp.float32),
                  jax.ShapeDtypeStruct((N,), jnp.float32)],
        mesh=_mesh,
        compiler_params=pltpu.CompilerParams(use_tc_tiling_on_sc=False),
        scratch_types=[
            pltpu.VMEM((CHUNK,), jnp.int32),
            pltpu.VMEM((CHUNK,), jnp.float32),
            pltpu.VMEM((SLICE,), jnp.float32),
            pltpu.VMEM_SHARED((N,), jnp.float32),
        ],
    )
    return f(col, ones128, zcol)


def _sc_agg(y, row, col, zslab):
    def body(y_hbm, row_hbm, col_hbm, z_hbm, u0_hbm, u1_hbm,
             ridx_v, cidx_v, rows_v, slab_v, acc_sh, sem):
        cid = lax.axis_index("c")
        sid = lax.axis_index("s")
        pltpu.sync_copy(z_hbm, slab_v)

        @pl.when(sid < NS - 1)
        def _():
            pltpu.sync_copy(slab_v, acc_sh.at[pl.ds(sid * SLICE, SLICE)])

        @pl.when(sid == NS - 1)
        def _():
            pltpu.sync_copy(slab_v.at[pl.ds(0, LAST_SLICE)],
                            acc_sh.at[pl.ds(sid * SLICE, LAST_SLICE)])

        plsc.subcore_barrier()
        ntrips = jnp.where(sid < EXTRA, BASE_TRIPS + 1, BASE_TRIPS)

        def trip(k, c):
            g = cid * CPC + sid + k * NS
            base = g * CHUNK
            pltpu.sync_copy(row_hbm.at[pl.ds(base, CHUNK)], ridx_v)
            pltpu.sync_copy(col_hbm.at[pl.ds(base, CHUNK)], cidx_v)
            pltpu.async_copy(y_hbm.at[ridx_v], rows_v, sem).wait()
            pltpu.sync_copy(rows_v, acc_sh.at[cidx_v], add=True)
            return c

        lax.fori_loop(0, ntrips, trip, 0)
        plsc.subcore_barrier()
        for c, dst in ((0, u0_hbm), (1, u1_hbm)):
            @pl.when(cid == c)
            def _(dst=dst):
                @pl.when(sid < NS - 1)
                def _():
                    pltpu.sync_copy(acc_sh.at[pl.ds(sid * SLICE, SLICE)],
                                    slab_v)
                    pltpu.sync_copy(slab_v, dst.at[pl.ds(sid * SLICE, SLICE)])

                @pl.when(sid == NS - 1)
                def _():
                    pltpu.sync_copy(acc_sh.at[pl.ds(sid * SLICE, LAST_SLICE)],
                                    slab_v.at[pl.ds(0, LAST_SLICE)])
                    pltpu.sync_copy(slab_v.at[pl.ds(0, LAST_SLICE)],
                                    dst.at[pl.ds(sid * SLICE, LAST_SLICE)])

    f = pl.kernel(
        body,
        out_type=[jax.ShapeDtypeStruct((N, H), jnp.float32),
                  jax.ShapeDtypeStruct((N, H), jnp.float32)],
        mesh=_mesh,
        compiler_params=pltpu.CompilerParams(use_tc_tiling_on_sc=False),
        scratch_types=[
            pltpu.VMEM((CHUNK,), jnp.int32),
            pltpu.VMEM((CHUNK,), jnp.int32),
            pltpu.VMEM((CHUNK, H), jnp.float32),
            pltpu.VMEM((SLICE, H), jnp.float32),
            pltpu.VMEM_SHARED((N, H), jnp.float32),
            pltpu.SemaphoreType.DMA,
        ],
    )
    return f(y, row, col, zslab)


def _tc_first(x, W1, d0, d1):
    def body(x_ref, w_ref, d0_ref, d1_ref, xw_ref, y_ref):
        deg = d0_ref[...] + d1_ref[...] + 1.0
        dinv = lax.rsqrt(deg)
        xw = jnp.dot(x_ref[...], w_ref[...], preferred_element_type=jnp.float32, precision=lax.Precision.HIGHEST)
        xw_ref[...] = xw
        y_ref[...] = xw * dinv

    return pl.pallas_call(
        body,
        compiler_params=pltpu.CompilerParams(vmem_limit_bytes=100 * 1024 * 1024),
        out_shape=[
            jax.ShapeDtypeStruct((N, H), jnp.float32),
            jax.ShapeDtypeStruct((N, H), jnp.float32),
        ],
    )(x, W1, d0, d1)


def _tc_mid(U0, U1, xw1, d0, d1, b1, W2):
    def body(u0_ref, u1_ref, xw1_ref, d0_ref, d1_ref, b_ref, w_ref,
             xw2_ref, y2_ref):
        deg = d0_ref[...] + d1_ref[...] + 1.0
        dinv = lax.rsqrt(deg)
        h = jnp.maximum(
            dinv * (u0_ref[...] + u1_ref[...]) + dinv * dinv * xw1_ref[...]
            + b_ref[...], 0.0)
        xw2 = jnp.dot(h, w_ref[...], preferred_element_type=jnp.float32, precision=lax.Precision.HIGHEST)
        xw2_ref[...] = xw2
        y2_ref[...] = xw2 * dinv

    return pl.pallas_call(
        body,
        compiler_params=pltpu.CompilerParams(vmem_limit_bytes=100 * 1024 * 1024),
        out_shape=[
            jax.ShapeDtypeStruct((N, H), jnp.float32),
            jax.ShapeDtypeStruct((N, H), jnp.float32),
        ],
    )(U0, U1, xw1, d0, d1, b1, W2)


def _tc_last(U0, U1, xw2, d0, d1, b2, batch2, Wf1, bf1, Wf2, bf2):
    def body(u0_ref, u1_ref, xw2_ref, d0_ref, d1_ref, b_ref, bat_ref,
             wf1_ref, bf1_ref, wf2_ref, bf2_ref, out_ref):
        deg = d0_ref[...] + d1_ref[...] + 1.0
        dinv = lax.rsqrt(deg)
        h = jnp.maximum(
            dinv * (u0_ref[...] + u1_ref[...]) + dinv * dinv * xw2_ref[...]
            + b_ref[...], 0.0)
        gids = lax.broadcasted_iota(jnp.int32, (1, G), 1)
        onehot = (bat_ref[...] == gids).astype(jnp.float32)
        dn = (((0,), (0,)), ((), ()))
        sums = lax.dot_general(onehot, h, dn, preferred_element_type=jnp.float32, precision=lax.Precision.HIGHEST)
        cnts = lax.dot_general(onehot, jnp.ones((N, 1), jnp.float32), dn,
                               preferred_element_type=jnp.float32, precision=lax.Precision.HIGHEST)
        p = sums / jnp.maximum(cnts, 1.0)
        q = jnp.maximum(
            jnp.dot(p, wf1_ref[...], preferred_element_type=jnp.float32, precision=lax.Precision.HIGHEST)
            + bf1_ref[...], 0.0)
        out_ref[...] = (
            jnp.dot(q, wf2_ref[...], preferred_element_type=jnp.float32, precision=lax.Precision.HIGHEST)
            + bf2_ref[...])

    return pl.pallas_call(
        body,
        compiler_params=pltpu.CompilerParams(vmem_limit_bytes=100 * 1024 * 1024),
        out_shape=jax.ShapeDtypeStruct((G, 1), jnp.float32),
    )(U0, U1, xw2, d0, d1, b2, batch2, Wf1, bf1, Wf2, bf2)


def kernel(x, edge_index, edge_attr, batch, W1, b1, W2, b2, Wf1, bf1, Wf2, bf2):
    del edge_attr
    row = edge_index[0]
    col = edge_index[1]
    ones128 = jnp.ones((CHUNK,), jnp.float32)
    zslab = jnp.zeros((SLICE, H), jnp.float32)
    zcol = jnp.zeros((SLICE,), jnp.float32)

    dp0, dp1 = _sc_deg(col, ones128, zcol)
    d0 = dp0.reshape(N, 1)
    d1 = dp1.reshape(N, 1)

    xw1, y1 = _tc_first(x, W1, d0, d1)
    U10, U11 = _sc_agg(y1, row, col, zslab)
    xw2, y2 = _tc_mid(U10, U11, xw1, d0, d1, b1.reshape(1, H), W2)
    U20, U21 = _sc_agg(y2, row, col, zslab)
    out = _tc_last(U20, U21, xw2, d0, d1, b2.reshape(1, H),
                   batch.reshape(N, 1), Wf1, bf1.reshape(1, H // 2),
                   Wf2, bf2.reshape(1, 1))
    return out


# speedup vs baseline: 24.4758x; 1.3598x over previous
import functools

import jax
import jax.numpy as jnp
from jax import lax
from jax.experimental import pallas as pl
from jax.experimental.pallas import tpu as pltpu
from jax.experimental.pallas import tpu_sc as plsc

N = 10000
E = 320000
DIN = 128
H = 64
G = 64

NC = 2
NS = 16
CHUNK = 128
NCHUNKS = E // CHUNK
CPC = NCHUNKS // NC
BASE_TRIPS = CPC // NS
EXTRA = CPC - BASE_TRIPS * NS
K = 2
NW = NC * NS
NBLOCKS = NCHUNKS // K
BASE_BLOCKS = NBLOCKS // NW
EXTRA_BLOCKS = NBLOCKS - BASE_BLOCKS * NW
MAX_BLOCKS = BASE_BLOCKS + 1
SLAB = 128
SLICE = 640
LAST_SLICE = N - (NS - 1) * SLICE

_mesh = plsc.VectorSubcoreMesh(core_axis_name="c", subcore_axis_name="s")


def _sc_deg(col, ones128, zcol):
    def body(col_hbm, ones_hbm, z_hbm, deg0_hbm, deg1_hbm, idx_v, ones_v,
             slab_v, acc_sh):
        cid = lax.axis_index("c")
        sid = lax.axis_index("s")
        pltpu.sync_copy(ones_hbm, ones_v)
        pltpu.sync_copy(z_hbm, slab_v)

        @pl.when(sid < NS - 1)
        def _():
            pltpu.sync_copy(slab_v, acc_sh.at[pl.ds(sid * SLICE, SLICE)])

        @pl.when(sid == NS - 1)
        def _():
            pltpu.sync_copy(slab_v.at[pl.ds(0, LAST_SLICE)],
                            acc_sh.at[pl.ds(sid * SLICE, LAST_SLICE)])

        plsc.subcore_barrier()
        ntrips = jnp.where(sid < EXTRA, BASE_TRIPS + 1, BASE_TRIPS)

        def trip(k, c):
            g = cid * CPC + sid + k * NS
            base = g * CHUNK
            pltpu.sync_copy(col_hbm.at[pl.ds(base, CHUNK)], idx_v)
            pltpu.sync_copy(ones_v, acc_sh.at[idx_v], add=True)
            return c

        lax.fori_loop(0, ntrips, trip, 0)
        plsc.subcore_barrier()
        for c, dst in ((0, deg0_hbm), (1, deg1_hbm)):
            @pl.when(cid == c)
            def _(dst=dst):
                @pl.when(sid < NS - 1)
                def _():
                    pltpu.sync_copy(acc_sh.at[pl.ds(sid * SLICE, SLICE)],
                                    slab_v)
                    pltpu.sync_copy(slab_v, dst.at[pl.ds(sid * SLICE, SLICE)])

                @pl.when(sid == NS - 1)
                def _():
                    pltpu.sync_copy(acc_sh.at[pl.ds(sid * SLICE, LAST_SLICE)],
                                    slab_v.at[pl.ds(0, LAST_SLICE)])
                    pltpu.sync_copy(slab_v.at[pl.ds(0, LAST_SLICE)],
                                    dst.at[pl.ds(sid * SLICE, LAST_SLICE)])

    f = pl.kernel(
        body,
        out_type=[jax.ShapeDtypeStruct((N,), jnp.float32),
                  jax.ShapeDtypeStruct((N,), jnp.float32)],
        mesh=_mesh,
        compiler_params=pltpu.CompilerParams(use_tc_tiling_on_sc=False),
        scratch_types=[
            pltpu.VMEM((CHUNK,), jnp.int32),
            pltpu.VMEM((CHUNK,), jnp.float32),
            pltpu.VMEM((SLICE,), jnp.float32),
            pltpu.VMEM_SHARED((N,), jnp.float32),
        ],
    )
    return f(col, ones128, zcol)


def _sc_agg(y, row2, col2, zslab):
    def body(y_hbm, row_hbm, col_hbm, z_hbm, u0_hbm, u1_hbm,
             ridx_a, cidx_a, rows_a, gs_a, ridx_b, cidx_b, rows_b, gs_b,
             slab_v, acc_sh):
        cid = lax.axis_index("c")
        sid = lax.axis_index("s")
        wid = cid * NS + sid
        pltpu.sync_copy(z_hbm, slab_v)

        @pl.when(sid < NS - 1)
        def _():
            for i in range(SLICE // SLAB):
                pltpu.sync_copy(slab_v,
                                acc_sh.at[pl.ds(sid * SLICE + i * SLAB, SLAB)])

        @pl.when(sid == NS - 1)
        def _():
            for i in range(LAST_SLICE // SLAB):
                pltpu.sync_copy(slab_v,
                                acc_sh.at[pl.ds(sid * SLICE + i * SLAB, SLAB)])
            rem = LAST_SLICE % SLAB
            if rem:
                pltpu.sync_copy(
                    slab_v.at[pl.ds(0, rem)],
                    acc_sh.at[pl.ds(sid * SLICE + LAST_SLICE - rem, rem)])

        plsc.subcore_barrier()
        ntrips = jnp.where(wid < EXTRA_BLOCKS, BASE_BLOCKS + 1, BASE_BLOCKS)

        def fire(t, ridx_v, cidx_v, rows_v, gsem):
            base = (wid + t * NW) * K * CHUNK
            for j in range(K):
                pltpu.sync_copy(row_hbm.at[pl.ds(base + j * CHUNK, CHUNK)],
                                ridx_v.at[j])
                pltpu.sync_copy(col_hbm.at[pl.ds(base + j * CHUNK, CHUNK)],
                                cidx_v.at[j])
            for j in range(K):
                pltpu.async_copy(y_hbm.at[ridx_v.at[j]], rows_v.at[j], gsem)

        def drain_scatter(ridx_v, cidx_v, rows_v, gsem):
            for j in range(K):
                pltpu.make_async_copy(y_hbm.at[pl.ds(0, CHUNK)],
                                      rows_v.at[j], gsem).wait()
            for j in range(K):
                pltpu.sync_copy(rows_v.at[j], acc_sh.at[cidx_v.at[j]], add=True)

        bufs = ((ridx_a, cidx_a, rows_a, gs_a), (ridx_b, cidx_b, rows_b, gs_b))

        @pl.when(ntrips > 0)
        def _():
            fire(0, *bufs[0])

        def pair(p, c):
            for half in (0, 1):
                t = 2 * p + half

                @pl.when(t < ntrips)
                def _(t=t, half=half):
                    @pl.when(t + 1 < ntrips)
                    def _(t=t, half=half):
                        fire(t + 1, *bufs[1 - half])
                    drain_scatter(*bufs[half])
            return c

        lax.fori_loop(0, (MAX_BLOCKS + 1) // 2, pair, 0)
        plsc.subcore_barrier()

        def put(dst, off, nrows):
            pltpu.sync_copy(acc_sh.at[pl.ds(off, nrows)],
                            slab_v.at[pl.ds(0, nrows)])
            pltpu.sync_copy(slab_v.at[pl.ds(0, nrows)],
                            dst.at[pl.ds(off, nrows)])

        for c, dst in ((0, u0_hbm), (1, u1_hbm)):
            @pl.when(cid == c)
            def _(dst=dst):
                @pl.when(sid < NS - 1)
                def _():
                    for i in range(SLICE // SLAB):
                        put(dst, sid * SLICE + i * SLAB, SLAB)

                @pl.when(sid == NS - 1)
                def _():
                    for i in range(LAST_SLICE // SLAB):
                        put(dst, sid * SLICE + i * SLAB, SLAB)
                    rem = LAST_SLICE % SLAB
                    if rem:
                        put(dst, sid * SLICE + LAST_SLICE - rem, rem)

    f = pl.kernel(
        body,
        out_type=[jax.ShapeDtypeStruct((N, H), jnp.float32),
                  jax.ShapeDtypeStruct((N, H), jnp.float32)],
        mesh=_mesh,
        compiler_params=pltpu.CompilerParams(use_tc_tiling_on_sc=False),
        scratch_types=[
            pltpu.VMEM((K, CHUNK), jnp.int32),
            pltpu.VMEM((K, CHUNK), jnp.int32),
            pltpu.VMEM((K, CHUNK, H), jnp.float32),
            pltpu.SemaphoreType.DMA,
            pltpu.VMEM((K, CHUNK), jnp.int32),
            pltpu.VMEM((K, CHUNK), jnp.int32),
            pltpu.VMEM((K, CHUNK, H), jnp.float32),
            pltpu.SemaphoreType.DMA,
            pltpu.VMEM((SLAB, H), jnp.float32),
            pltpu.VMEM_SHARED((N, H), jnp.float32),
        ],
    )
    return f(y, row2, col2, zslab)


def _tc_first(x, W1, d0, d1):
    def body(x_ref, w_ref, d0_ref, d1_ref, xw_ref, y_ref):
        deg = d0_ref[...] + d1_ref[...] + 1.0
        dinv = lax.rsqrt(deg)
        xw = jnp.dot(x_ref[...], w_ref[...], preferred_element_type=jnp.float32, precision=lax.Precision.HIGHEST)
        xw_ref[...] = xw
        y_ref[...] = xw * dinv

    return pl.pallas_call(
        body,
        compiler_params=pltpu.CompilerParams(vmem_limit_bytes=100 * 1024 * 1024),
        out_shape=[
            jax.ShapeDtypeStruct((N, H), jnp.float32),
            jax.ShapeDtypeStruct((N, H), jnp.float32),
        ],
    )(x, W1, d0, d1)


def _tc_mid(U0, U1, xw1, d0, d1, b1, W2):
    def body(u0_ref, u1_ref, xw1_ref, d0_ref, d1_ref, b_ref, w_ref,
             xw2_ref, y2_ref):
        deg = d0_ref[...] + d1_ref[...] + 1.0
        dinv = lax.rsqrt(deg)
        h = jnp.maximum(
            dinv * (u0_ref[...] + u1_ref[...]) + dinv * dinv * xw1_ref[...]
            + b_ref[...], 0.0)
        xw2 = jnp.dot(h, w_ref[...], preferred_element_type=jnp.float32, precision=lax.Precision.HIGHEST)
        xw2_ref[...] = xw2
        y2_ref[...] = xw2 * dinv

    return pl.pallas_call(
        body,
        compiler_params=pltpu.CompilerParams(vmem_limit_bytes=100 * 1024 * 1024),
        out_shape=[
            jax.ShapeDtypeStruct((N, H), jnp.float32),
            jax.ShapeDtypeStruct((N, H), jnp.float32),
        ],
    )(U0, U1, xw1, d0, d1, b1, W2)


def _tc_last(U0, U1, xw2, d0, d1, b2, batch2, Wf1, bf1, Wf2, bf2):
    def body(u0_ref, u1_ref, xw2_ref, d0_ref, d1_ref, b_ref, bat_ref,
             wf1_ref, bf1_ref, wf2_ref, bf2_ref, out_ref):
        deg = d0_ref[...] + d1_ref[...] + 1.0
        dinv = lax.rsqrt(deg)
        h = jnp.maximum(
            dinv * (u0_ref[...] + u1_ref[...]) + dinv * dinv * xw2_ref[...]
            + b_ref[...], 0.0)
        gids = lax.broadcasted_iota(jnp.int32, (1, G), 1)
        onehot = (bat_ref[...] == gids).astype(jnp.float32)
        dn = (((0,), (0,)), ((), ()))
        sums = lax.dot_general(onehot, h, dn, preferred_element_type=jnp.float32, precision=lax.Precision.HIGHEST)
        cnts = lax.dot_general(onehot, jnp.ones((N, 1), jnp.float32), dn,
                               preferred_element_type=jnp.float32, precision=lax.Precision.HIGHEST)
        p = sums / jnp.maximum(cnts, 1.0)
        q = jnp.maximum(
            jnp.dot(p, wf1_ref[...], preferred_element_type=jnp.float32, precision=lax.Precision.HIGHEST)
            + bf1_ref[...], 0.0)
        out_ref[...] = (
            jnp.dot(q, wf2_ref[...], preferred_element_type=jnp.float32, precision=lax.Precision.HIGHEST)
            + bf2_ref[...])

    return pl.pallas_call(
        body,
        compiler_params=pltpu.CompilerParams(vmem_limit_bytes=100 * 1024 * 1024),
        out_shape=jax.ShapeDtypeStruct((G, 1), jnp.float32),
    )(U0, U1, xw2, d0, d1, b2, batch2, Wf1, bf1, Wf2, bf2)


def kernel(x, edge_index, edge_attr, batch, W1, b1, W2, b2, Wf1, bf1, Wf2, bf2):
    del edge_attr
    row = edge_index[0]
    col = edge_index[1]
    ones128 = jnp.ones((CHUNK,), jnp.float32)
    zslab = jnp.zeros((SLAB, H), jnp.float32)
    zcol = jnp.zeros((SLICE,), jnp.float32)

    dp0, dp1 = _sc_deg(col, ones128, zcol)
    d0 = dp0.reshape(N, 1)
    d1 = dp1.reshape(N, 1)

    xw1, y1 = _tc_first(x, W1, d0, d1)
    U10, U11 = _sc_agg(y1, row, col, zslab)
    xw2, y2 = _tc_mid(U10, U11, xw1, d0, d1, b1.reshape(1, H), W2)
    U20, U21 = _sc_agg(y2, row, col, zslab)
    out = _tc_last(U20, U21, xw2, d0, d1, b2.reshape(1, H),
                   batch.reshape(N, 1), Wf1, bf1.reshape(1, H // 2),
                   Wf2, bf2.reshape(1, 1))
    return out
